# baseline (speedup 1.0000x reference)
import functools

import jax
import jax.numpy as jnp
from jax import lax
from jax.experimental import pallas as pl
from jax.experimental.pallas import tpu as pltpu
from jax.experimental.pallas import tpu_sc as plsc

N = 100000
E = 1600000
DIM = 32
NF = 11
EF = 4

NC = 2
NS = 16
NW = NC * NS

E_PAD = 1638400
CH = 1024
EPW = E_PAD // NW
EPS = E_PAD // NS
NQ = N // 4
SH = NQ + 8
TRASH = NQ

_mesh = plsc.VectorSubcoreMesh(core_axis_name="c", subcore_axis_name="s",
                               num_cores=NC, num_subcores=NS)


def _sc_gather(table, idx2d, F):
    @functools.partial(
        pl.kernel, mesh=_mesh,
        out_type=jax.ShapeDtypeStruct((E_PAD, F), jnp.float32),
        compiler_params=pltpu.CompilerParams(use_tc_tiling_on_sc=False),
        scratch_types=[
            pltpu.VMEM((8, 128), jnp.int32),
            pltpu.VMEM((CH, F), jnp.float32),
            pltpu.SemaphoreType.DMA,
        ],
    )
    def k(table_hbm, idx_hbm, out_hbm, idx_v, rows_v, sem):
        wid = lax.axis_index("s") * NC + lax.axis_index("c")
        nch = EPW // CH

        def body(i, _):
            e0 = wid * EPW + i * CH
            r0 = pl.multiple_of(e0 // 128, 8)
            pltpu.sync_copy(idx_hbm.at[pl.ds(r0, 8)], idx_v)
            descs = []
            for j in range(8):
                descs.append(pltpu.async_copy(
                    table_hbm.at[idx_v.at[j]],
                    rows_v.at[pl.ds(j * 128, 128)], sem))
            for d in descs:
                d.wait()
            pltpu.sync_copy(rows_v, out_hbm.at[pl.ds(e0, CH)])
            return 0

        lax.fori_loop(0, nch, body, 0)

    return k(table, idx2d)


def _sc_scatter(msg, dst2d):
    @functools.partial(
        pl.kernel, mesh=_mesh,
        out_type=jax.ShapeDtypeStruct((N, DIM), jnp.float32),
        compiler_params=pltpu.CompilerParams(use_tc_tiling_on_sc=False),
        scratch_types=[
            pltpu.VMEM((8, 128), jnp.int32),
            pltpu.VMEM((8, 128), jnp.int32),
            pltpu.VMEM((CH, DIM), jnp.float32),
            pltpu.VMEM_SHARED((SH, DIM), jnp.float32),
            pltpu.SemaphoreType.DMA,
        ],
    )
    def k(msg_hbm, dst_hbm, out_hbm, raw_v, idx_v, val_v, shared, sem):
        c = lax.axis_index("c")
        s = lax.axis_index("s")

        zer = jnp.zeros((16,), jnp.float32)

        def zfill(i, _):
            val_v[i, pl.ds(0, 16)] = zer
            val_v[i, pl.ds(16, 16)] = zer
            return 0

        for p in range(2):
            lax.fori_loop(0, CH, zfill, 0)
            z0 = s * (SH // NS)
            pltpu.sync_copy(val_v, shared.at[pl.ds(z0, CH)])
            pltpu.sync_copy(val_v.at[pl.ds(0, SH // NS - CH)],
                            shared.at[pl.ds(z0 + CH, SH // NS - CH)])
            plsc.subcore_barrier()

            lo = (2 * p + c) * NQ

            def body2(i, _):
                e0 = s * EPS + i * CH
                r0 = pl.multiple_of(e0 // 128, 8)
                pltpu.sync_copy(dst_hbm.at[pl.ds(r0, 8)], raw_v)
                for j in range(8):
                    for m in range(8):
                        v = raw_v[j, pl.ds(m * 16, 16)]
                        rel = v - lo
                        ok = (rel >= 0) & (rel < NQ)
                        idx_v[j, pl.ds(m * 16, 16)] = jnp.where(ok, rel, TRASH)
                pltpu.sync_copy(msg_hbm.at[pl.ds(e0, CH)], val_v)
                for j in range(8):
                    pltpu.sync_copy(val_v.at[pl.ds(j * 128, 128)],
                                    shared.at[idx_v.at[j]], add=True)
                return 0

            lax.fori_loop(0, EPS // CH, body2, 0)
            plsc.subcore_barrier()

            rows = NQ // NS
            pltpu.sync_copy(shared.at[pl.ds(s * rows, rows)],
                            out_hbm.at[pl.ds(lo + s * rows, rows)])

            @pl.when(s == 0)
            def _():
                pltpu.sync_copy(shared.at[pl.ds(NS * rows, NQ - NS * rows)],
                                out_hbm.at[pl.ds(lo + NS * rows, NQ - NS * rows)])

            plsc.subcore_barrier()

    return k(msg, dst2d)


def _tc_edge(ea_p, xs, w1, b1, w2p, b2p, inp):
    T = 1024
    G = E_PAD // T

    def body(ea_ref, xs_ref, w1_ref, b1_ref, w2p_ref, b2p_ref, out_ref):
        ea = ea_ref[...]
        xsv = xs_ref[...]
        h = jnp.maximum(
            jnp.dot(ea, w1_ref[...], preferred_element_type=jnp.float32)
            + b1_ref[...], 0.0)
        hq = jnp.broadcast_to(h[:, :, None], (T, DIM, inp)).reshape(T, DIM * inp)
        xq = jnp.broadcast_to(xsv[:, None, :], (T, DIM, inp)).reshape(T, DIM * inp)
        q = hq * xq
        out_ref[...] = (
            jnp.dot(q, w2p_ref[...], preferred_element_type=jnp.float32)
            + jnp.dot(xsv, b2p_ref[...], preferred_element_type=jnp.float32))

    return pl.pallas_call(
        body,
        grid=(G,),
        in_specs=[
            pl.BlockSpec((T, EF), lambda i: (i, 0)),
            pl.BlockSpec((T, inp), lambda i: (i, 0)),
            pl.BlockSpec((EF, DIM), lambda i: (0, 0)),
            pl.BlockSpec((1, DIM), lambda i: (0, 0)),
            pl.BlockSpec((DIM * inp, DIM), lambda i: (0, 0)),
            pl.BlockSpec((inp, DIM), lambda i: (0, 0)),
        ],
        out_specs=pl.BlockSpec((T, DIM), lambda i: (i, 0)),
        out_shape=jax.ShapeDtypeStruct((E_PAD, DIM), jnp.float32),
    )(ea_p, xs, w1, b1, w2p, b2p)


def _tc_node(agg, xtab, rootp, bias, inp):
    T = 1000
    G = N // T

    def body(agg_ref, x_ref, r_ref, b_ref, out_ref):
        out_ref[...] = jnp.maximum(
            agg_ref[...]
            + jnp.dot(x_ref[...], r_ref[...], preferred_element_type=jnp.float32)
            + b_ref[...], 0.0)

    return pl.pallas_call(
        body,
        grid=(G,),
        in_specs=[
            pl.BlockSpec((T, DIM), lambda i: (i, 0)),
            pl.BlockSpec((T, inp), lambda i: (i, 0)),
            pl.BlockSpec((inp, DIM), lambda i: (0, 0)),
            pl.BlockSpec((1, DIM), lambda i: (0, 0)),
        ],
        out_specs=pl.BlockSpec((T, DIM), lambda i: (i, 0)),
        out_shape=jax.ShapeDtypeStruct((N, DIM), jnp.float32),
    )(agg, xtab, rootp, bias)


def kernel(x, edge_index, edge_attr, batch,
           mlp1_w1, mlp1_b1, mlp1_w2, mlp1_b2, root1, bias1,
           mlp2_w1, mlp2_b1, mlp2_w2, mlp2_b2, root2, bias2):
    del batch
    src = edge_index[0].astype(jnp.int32)
    dst = edge_index[1].astype(jnp.int32)
    src2d = jnp.pad(src, (0, E_PAD - E)).reshape(E_PAD // 128, 128)
    dst2d = jnp.pad(dst, (0, E_PAD - E),
                    constant_values=N).reshape(E_PAD // 128, 128)
    ea_p = jnp.pad(edge_attr, ((0, E_PAD - E), (0, 0)))

    xpad = jnp.pad(x, ((0, 0), (0, 16 - NF)))
    w2r1 = mlp1_w2.reshape(DIM, NF, DIM)
    w2p1 = jnp.pad(w2r1, ((0, 0), (0, 16 - NF), (0, 0))).reshape(DIM * 16, DIM)
    b2p1 = jnp.pad(mlp1_b2.reshape(NF, DIM), ((0, 16 - NF), (0, 0)))
    root1p = jnp.pad(root1, ((0, 16 - NF), (0, 0)))

    w2p2 = mlp2_w2.reshape(DIM * DIM, DIM)
    b2p2 = mlp2_b2.reshape(DIM, DIM)

    def layer(xtab, w1, b1, w2p, b2p, rootp, bias, inp):
        xs = _sc_gather(xtab, src2d, inp)
        msg = _tc_edge(ea_p, xs, w1, b1.reshape(1, DIM), w2p, b2p, inp)
        agg = _sc_scatter(msg, dst2d)
        return _tc_node(agg, xtab, rootp, bias.reshape(1, DIM), inp)

    x1 = layer(xpad, mlp1_w1, mlp1_b1, w2p1, b2p1, root1p, bias1, 16)
    x2 = layer(x1, mlp2_w1, mlp2_b1, w2p2, b2p2, root2, bias2, DIM)
    return x2


# speedup vs baseline: 2.6940x; 2.6940x over previous
import functools

import jax
import jax.numpy as jnp
from jax import lax
from jax.experimental import pallas as pl
from jax.experimental.pallas import tpu as pltpu
from jax.experimental.pallas import tpu_sc as plsc

N = 100000
E = 1600000
DIM = 32
NF = 11
EF = 4

NC = 2
NS = 16
NW = NC * NS

E_PAD = 1638400
CH = 1024
EPW = E_PAD // NW
EPS = E_PAD // NS
NQ = N // 4
SH = NQ + 8
TRASH = NQ

_mesh = plsc.VectorSubcoreMesh(core_axis_name="c", subcore_axis_name="s",
                               num_cores=NC, num_subcores=NS)


def _sc_gather(table, idx2d, F):
    @functools.partial(
        pl.kernel, mesh=_mesh,
        out_type=jax.ShapeDtypeStruct((E_PAD, F), jnp.float32),
        compiler_params=pltpu.CompilerParams(use_tc_tiling_on_sc=False),
        scratch_types=[
            pltpu.VMEM((8, 128), jnp.int32),
            pltpu.VMEM((CH, F), jnp.float32),
            pltpu.SemaphoreType.DMA,
        ],
    )
    def k(table_hbm, idx_hbm, out_hbm, idx_v, rows_v, sem):
        wid = lax.axis_index("s") * NC + lax.axis_index("c")
        nch = EPW // CH

        def body(i, _):
            e0 = wid * EPW + i * CH
            r0 = pl.multiple_of(e0 // 128, 8)
            pltpu.sync_copy(idx_hbm.at[pl.ds(r0, 8)], idx_v)
            descs = []
            for j in range(8):
                descs.append(pltpu.async_copy(
                    table_hbm.at[idx_v.at[j]],
                    rows_v.at[pl.ds(j * 128, 128)], sem))
            for d in descs:
                d.wait()
            pltpu.sync_copy(rows_v, out_hbm.at[pl.ds(e0, CH)])
            return 0

        lax.fori_loop(0, nch, body, 0)

    return k(table, idx2d)


def _sc_scatter(msg, dst2d):
    @functools.partial(
        pl.kernel, mesh=_mesh,
        out_type=jax.ShapeDtypeStruct((N, DIM), jnp.float32),
        compiler_params=pltpu.CompilerParams(use_tc_tiling_on_sc=False),
        scratch_types=[
            pltpu.VMEM((8, 128), jnp.int32),
            pltpu.VMEM((8, 128), jnp.int32),
            pltpu.VMEM((CH, DIM), jnp.float32),
            pltpu.VMEM_SHARED((SH, DIM), jnp.float32),
            pltpu.SemaphoreType.DMA,
        ],
    )
    def k(msg_hbm, dst_hbm, out_hbm, raw_v, idx_v, val_v, shared, sem):
        c = lax.axis_index("c")
        s = lax.axis_index("s")

        zer = jnp.zeros((16,), jnp.float32)

        def zfill(i, _):
            val_v[i, pl.ds(0, 16)] = zer
            val_v[i, pl.ds(16, 16)] = zer
            return 0

        for p in range(2):
            lax.fori_loop(0, CH, zfill, 0)
            z0 = s * (SH // NS)
            pltpu.sync_copy(val_v, shared.at[pl.ds(z0, CH)])
            pltpu.sync_copy(val_v.at[pl.ds(0, SH // NS - CH)],
                            shared.at[pl.ds(z0 + CH, SH // NS - CH)])
            plsc.subcore_barrier()

            lo = (2 * p + c) * NQ

            def body2(i, _):
                e0 = s * EPS + i * CH
                r0 = pl.multiple_of(e0 // 128, 8)
                pltpu.sync_copy(dst_hbm.at[pl.ds(r0, 8)], raw_v)
                for j in range(8):
                    for m in range(8):
                        v = raw_v[j, pl.ds(m * 16, 16)]
                        rel = v - lo
                        ok = (rel >= 0) & (rel < NQ)
                        idx_v[j, pl.ds(m * 16, 16)] = jnp.where(ok, rel, TRASH)
                pltpu.sync_copy(msg_hbm.at[pl.ds(e0, CH)], val_v)
                for j in range(8):
                    pltpu.sync_copy(val_v.at[pl.ds(j * 128, 128)],
                                    shared.at[idx_v.at[j]], add=True)
                return 0

            lax.fori_loop(0, EPS // CH, body2, 0)
            plsc.subcore_barrier()

            rows = NQ // NS
            pltpu.sync_copy(shared.at[pl.ds(s * rows, rows)],
                            out_hbm.at[pl.ds(lo + s * rows, rows)])

            @pl.when(s == 0)
            def _():
                pltpu.sync_copy(shared.at[pl.ds(NS * rows, NQ - NS * rows)],
                                out_hbm.at[pl.ds(lo + NS * rows, NQ - NS * rows)])

            plsc.subcore_barrier()

    return k(msg, dst2d)


def _tc_edge(ea_p, xs, w1, b1, w2p, b2p, rmat, smat, inp):
    T = 1024
    G = E_PAD // T

    def body(ea_ref, xs_ref, w1_ref, b1_ref, w2p_ref, b2p_ref, r_ref, s_ref,
             out_ref):
        ea = ea_ref[...]
        xsv = xs_ref[...]
        h = jnp.maximum(
            jnp.dot(ea, w1_ref[...], preferred_element_type=jnp.float32)
            + b1_ref[...], 0.0)
        hq = jnp.dot(h, r_ref[...], preferred_element_type=jnp.float32)
        xq = jnp.dot(xsv, s_ref[...], preferred_element_type=jnp.float32)
        q = hq * xq
        out_ref[...] = (
            jnp.dot(q, w2p_ref[...], preferred_element_type=jnp.float32)
            + jnp.dot(xsv, b2p_ref[...], preferred_element_type=jnp.float32))

    return pl.pallas_call(
        body,
        grid=(G,),
        in_specs=[
            pl.BlockSpec((T, EF), lambda i: (i, 0)),
            pl.BlockSpec((T, inp), lambda i: (i, 0)),
            pl.BlockSpec((EF, DIM), lambda i: (0, 0)),
            pl.BlockSpec((1, DIM), lambda i: (0, 0)),
            pl.BlockSpec((DIM * inp, DIM), lambda i: (0, 0)),
            pl.BlockSpec((inp, DIM), lambda i: (0, 0)),
            pl.BlockSpec((DIM, DIM * inp), lambda i: (0, 0)),
            pl.BlockSpec((inp, DIM * inp), lambda i: (0, 0)),
        ],
        out_specs=pl.BlockSpec((T, DIM), lambda i: (i, 0)),
        out_shape=jax.ShapeDtypeStruct((E_PAD, DIM), jnp.float32),
    )(ea_p, xs, w1, b1, w2p, b2p, rmat, smat)


def _tc_node(agg, xtab, rootp, bias, inp):
    T = 1000
    G = N // T

    def body(agg_ref, x_ref, r_ref, b_ref, out_ref):
        out_ref[...] = jnp.maximum(
            agg_ref[...]
            + jnp.dot(x_ref[...], r_ref[...], preferred_element_type=jnp.float32)
            + b_ref[...], 0.0)

    return pl.pallas_call(
        body,
        grid=(G,),
        in_specs=[
            pl.BlockSpec((T, DIM), lambda i: (i, 0)),
            pl.BlockSpec((T, inp), lambda i: (i, 0)),
            pl.BlockSpec((inp, DIM), lambda i: (0, 0)),
            pl.BlockSpec((1, DIM), lambda i: (0, 0)),
        ],
        out_specs=pl.BlockSpec((T, DIM), lambda i: (i, 0)),
        out_shape=jax.ShapeDtypeStruct((N, DIM), jnp.float32),
    )(agg, xtab, rootp, bias)


def kernel(x, edge_index, edge_attr, batch,
           mlp1_w1, mlp1_b1, mlp1_w2, mlp1_b2, root1, bias1,
           mlp2_w1, mlp2_b1, mlp2_w2, mlp2_b2, root2, bias2):
    del batch
    src = edge_index[0].astype(jnp.int32)
    dst = edge_index[1].astype(jnp.int32)
    src2d = jnp.pad(src, (0, E_PAD - E)).reshape(E_PAD // 128, 128)
    dst2d = jnp.pad(dst, (0, E_PAD - E),
                    constant_values=N).reshape(E_PAD // 128, 128)
    ea_p = jnp.pad(edge_attr, ((0, E_PAD - E), (0, 0)))

    xpad = jnp.pad(x, ((0, 0), (0, 16 - NF)))
    w2r1 = mlp1_w2.reshape(DIM, NF, DIM)
    w2p1 = jnp.pad(w2r1, ((0, 0), (0, 16 - NF), (0, 0))).reshape(DIM * 16, DIM)
    b2p1 = jnp.pad(mlp1_b2.reshape(NF, DIM), ((0, 16 - NF), (0, 0)))
    root1p = jnp.pad(root1, ((0, 16 - NF), (0, 0)))

    w2p2 = mlp2_w2.reshape(DIM * DIM, DIM)
    b2p2 = mlp2_b2.reshape(DIM, DIM)

    def layer(xtab, w1, b1, w2p, b2p, rootp, bias, inp):
        rmat = jnp.repeat(jnp.eye(DIM, dtype=jnp.float32), inp, axis=1)
        smat = jnp.tile(jnp.eye(inp, dtype=jnp.float32), (1, DIM))
        xs = _sc_gather(xtab, src2d, inp)
        msg = _tc_edge(ea_p, xs, w1, b1.reshape(1, DIM), w2p, b2p,
                       rmat, smat, inp)
        agg = _sc_scatter(msg, dst2d)
        return _tc_node(agg, xtab, rootp, bias.reshape(1, DIM), inp)

    x1 = layer(xpad, mlp1_w1, mlp1_b1, w2p1, b2p1, root1p, bias1, 16)
    x2 = layer(x1, mlp2_w1, mlp2_b1, w2p2, b2p2, root2, bias2, DIM)
    return x2


# speedup vs baseline: 2.7159x; 1.0082x over previous
import functools

import jax
import jax.numpy as jnp
from jax import lax
from jax.experimental import pallas as pl
from jax.experimental.pallas import tpu as pltpu
from jax.experimental.pallas import tpu_sc as plsc

N = 100000
E = 1600000
DIM = 32
NF = 11
EF = 4

NC = 2
NS = 16
NW = NC * NS

E_PAD = 1638400
CH = 1024
EPW = E_PAD // NW
EPS = E_PAD // NS
NQ = N // 4
SH = NQ + 8
TRASH = NQ

_mesh = plsc.VectorSubcoreMesh(core_axis_name="c", subcore_axis_name="s",
                               num_cores=NC, num_subcores=NS)


def _sc_gather(table, idx2d, F):
    @functools.partial(
        pl.kernel, mesh=_mesh,
        out_type=jax.ShapeDtypeStruct((E_PAD, F), jnp.float32),
        compiler_params=pltpu.CompilerParams(use_tc_tiling_on_sc=False),
        scratch_types=[
            pltpu.VMEM((2, 8, 128), jnp.int32),
            pltpu.VMEM((2, CH, F), jnp.float32),
            pltpu.SemaphoreType.DMA,
            pltpu.SemaphoreType.DMA,
            pltpu.SemaphoreType.DMA,
            pltpu.SemaphoreType.DMA,
            pltpu.SemaphoreType.DMA,
        ],
    )
    def k(table_hbm, idx_hbm, out_hbm, idx_v, rows_v, gsem,
          isem0, isem1, wsem0, wsem1):
        isem = [isem0, isem1]
        wsem = [wsem0, wsem1]
        wid = lax.axis_index("s") * NC + lax.axis_index("c")
        nch = EPW // CH

        def start_idx(ci, b):
            e0 = wid * EPW + ci * CH
            r0 = pl.multiple_of(e0 // 128, 8)
            pltpu.async_copy(idx_hbm.at[pl.ds(r0, 8)], idx_v.at[b], isem[b])

        def drain_idx(b):
            pltpu.make_async_copy(idx_hbm.at[pl.ds(0, 8)],
                                  idx_v.at[b], isem[b]).wait()

        def drain_out(b):
            pltpu.make_async_copy(rows_v.at[b],
                                  out_hbm.at[pl.ds(0, CH)], wsem[b]).wait()

        start_idx(0, 0)
        start_idx(1, 1)

        def body(g, _):
            for b in range(2):
                ci = g * 2 + b
                e0 = wid * EPW + ci * CH
                drain_idx(b)
                @pl.when(ci >= 2)
                def _():
                    drain_out(b)
                descs = []
                for j in range(8):
                    descs.append(pltpu.async_copy(
                        table_hbm.at[idx_v.at[b, j]],
                        rows_v.at[b, pl.ds(j * 128, 128)], gsem))
                for d in descs:
                    d.wait()
                pltpu.async_copy(rows_v.at[b], out_hbm.at[pl.ds(e0, CH)],
                                 wsem[b])

                @pl.when(ci + 2 < nch)
                def _():
                    start_idx(ci + 2, b)
            return 0

        lax.fori_loop(0, nch // 2, body, 0)
        drain_out(0)
        drain_out(1)

    return k(table, idx2d)


def _sc_scatter(msg, dst2d):
    @functools.partial(
        pl.kernel, mesh=_mesh,
        out_type=jax.ShapeDtypeStruct((N, DIM), jnp.float32),
        compiler_params=pltpu.CompilerParams(use_tc_tiling_on_sc=False),
        scratch_types=[
            pltpu.VMEM((2, 8, 128), jnp.int32),
            pltpu.VMEM((8, 128), jnp.int32),
            pltpu.VMEM((2, CH, DIM), jnp.float32),
            pltpu.VMEM_SHARED((SH, DIM), jnp.float32),
            pltpu.SemaphoreType.DMA,
            pltpu.SemaphoreType.DMA,
            pltpu.SemaphoreType.DMA,
            pltpu.SemaphoreType.DMA,
            pltpu.SemaphoreType.DMA,
        ],
    )
    def k(msg_hbm, dst_hbm, out_hbm, raw_v, idx_v, val_v, shared,
          ssem, rsem0, rsem1, msem0, msem1):
        c = lax.axis_index("c")
        s = lax.axis_index("s")
        rsem = [rsem0, rsem1]
        msem = [msem0, msem1]
        nchp = EPS // CH

        zer = jnp.zeros((16,), jnp.float32)

        def zfill(i, _):
            val_v[0, i, pl.ds(0, 16)] = zer
            val_v[0, i, pl.ds(16, 16)] = zer
            return 0

        def start_loads(ci, b):
            e0 = s * EPS + ci * CH
            r0 = pl.multiple_of(e0 // 128, 8)
            pltpu.async_copy(dst_hbm.at[pl.ds(r0, 8)], raw_v.at[b], rsem[b])
            pltpu.async_copy(msg_hbm.at[pl.ds(e0, CH)], val_v.at[b], msem[b])

        def drain_loads(b):
            pltpu.make_async_copy(dst_hbm.at[pl.ds(0, 8)],
                                  raw_v.at[b], rsem[b]).wait()
            pltpu.make_async_copy(msg_hbm.at[pl.ds(0, CH)],
                                  val_v.at[b], msem[b]).wait()

        for p in range(2):
            lax.fori_loop(0, CH, zfill, 0)
            z0 = s * (SH // NS)
            pltpu.sync_copy(val_v.at[0], shared.at[pl.ds(z0, CH)])
            pltpu.sync_copy(val_v.at[0, pl.ds(0, SH // NS - CH)],
                            shared.at[pl.ds(z0 + CH, SH // NS - CH)])
            plsc.subcore_barrier()

            lo = (2 * p + c) * NQ
            start_loads(0, 0)
            start_loads(1, 1)

            def body2(g, _):
                for b in range(2):
                    ci = g * 2 + b
                    drain_loads(b)
                    for j in range(8):
                        for m in range(8):
                            v = raw_v[b, j, pl.ds(m * 16, 16)]
                            rel = v - lo
                            ok = (rel >= 0) & (rel < NQ)
                            idx_v[j, pl.ds(m * 16, 16)] = jnp.where(
                                ok, rel, TRASH)
                    descs = []
                    for j in range(8):
                        descs.append(pltpu.async_copy(
                            val_v.at[b, pl.ds(j * 128, 128)],
                            shared.at[idx_v.at[j]], ssem, add=True))
                    for d in descs:
                        d.wait()

                    @pl.when(ci + 2 < nchp)
                    def _():
                        start_loads(ci + 2, b)
                return 0

            lax.fori_loop(0, nchp // 2, body2, 0)
            plsc.subcore_barrier()

            rows = NQ // NS
            pltpu.sync_copy(shared.at[pl.ds(s * rows, rows)],
                            out_hbm.at[pl.ds(lo + s * rows, rows)])

            @pl.when(s == 0)
            def _():
                pltpu.sync_copy(shared.at[pl.ds(NS * rows, NQ - NS * rows)],
                                out_hbm.at[pl.ds(lo + NS * rows, NQ - NS * rows)])

            plsc.subcore_barrier()

    return k(msg, dst2d)


def _tc_edge(ea_p, xs, w1, b1, w2p, b2p, rmat, smat, inp):
    T = 1024
    G = E_PAD // T

    def body(ea_ref, xs_ref, w1_ref, b1_ref, w2p_ref, b2p_ref, r_ref, s_ref,
             out_ref):
        ea = ea_ref[...]
        xsv = xs_ref[...]
        h = jnp.maximum(
            jnp.dot(ea, w1_ref[...], preferred_element_type=jnp.float32)
            + b1_ref[...], 0.0)
        hq = jnp.dot(h, r_ref[...], preferred_element_type=jnp.float32)
        xq = jnp.dot(xsv, s_ref[...], preferred_element_type=jnp.float32)
        q = hq * xq
        out_ref[...] = (
            jnp.dot(q, w2p_ref[...], preferred_element_type=jnp.float32)
            + jnp.dot(xsv, b2p_ref[...], preferred_element_type=jnp.float32))

    return pl.pallas_call(
        body,
        grid=(G,),
        in_specs=[
            pl.BlockSpec((T, EF), lambda i: (i, 0)),
            pl.BlockSpec((T, inp), lambda i: (i, 0)),
            pl.BlockSpec((EF, DIM), lambda i: (0, 0)),
            pl.BlockSpec((1, DIM), lambda i: (0, 0)),
            pl.BlockSpec((DIM * inp, DIM), lambda i: (0, 0)),
            pl.BlockSpec((inp, DIM), lambda i: (0, 0)),
            pl.BlockSpec((DIM, DIM * inp), lambda i: (0, 0)),
            pl.BlockSpec((inp, DIM * inp), lambda i: (0, 0)),
        ],
        out_specs=pl.BlockSpec((T, DIM), lambda i: (i, 0)),
        out_shape=jax.ShapeDtypeStruct((E_PAD, DIM), jnp.float32),
    )(ea_p, xs, w1, b1, w2p, b2p, rmat, smat)


def _tc_node(agg, xtab, rootp, bias, inp):
    T = 1000
    G = N // T

    def body(agg_ref, x_ref, r_ref, b_ref, out_ref):
        out_ref[...] = jnp.maximum(
            agg_ref[...]
            + jnp.dot(x_ref[...], r_ref[...], preferred_element_type=jnp.float32)
            + b_ref[...], 0.0)

    return pl.pallas_call(
        body,
        grid=(G,),
        in_specs=[
            pl.BlockSpec((T, DIM), lambda i: (i, 0)),
            pl.BlockSpec((T, inp), lambda i: (i, 0)),
            pl.BlockSpec((inp, DIM), lambda i: (0, 0)),
            pl.BlockSpec((1, DIM), lambda i: (0, 0)),
        ],
        out_specs=pl.BlockSpec((T, DIM), lambda i: (i, 0)),
        out_shape=jax.ShapeDtypeStruct((N, DIM), jnp.float32),
    )(agg, xtab, rootp, bias)


def kernel(x, edge_index, edge_attr, batch,
           mlp1_w1, mlp1_b1, mlp1_w2, mlp1_b2, root1, bias1,
           mlp2_w1, mlp2_b1, mlp2_w2, mlp2_b2, root2, bias2):
    del batch
    src = edge_index[0].astype(jnp.int32)
    dst = edge_index[1].astype(jnp.int32)
    src2d = jnp.pad(src, (0, E_PAD - E)).reshape(E_PAD // 128, 128)
    dst2d = jnp.pad(dst, (0, E_PAD - E),
                    constant_values=N).reshape(E_PAD // 128, 128)
    ea_p = jnp.pad(edge_attr, ((0, E_PAD - E), (0, 0)))

    xpad = jnp.pad(x, ((0, 0), (0, 16 - NF)))
    w2r1 = mlp1_w2.reshape(DIM, NF, DIM)
    w2p1 = jnp.pad(w2r1, ((0, 0), (0, 16 - NF), (0, 0))).reshape(DIM * 16, DIM)
    b2p1 = jnp.pad(mlp1_b2.reshape(NF, DIM), ((0, 16 - NF), (0, 0)))
    root1p = jnp.pad(root1, ((0, 16 - NF), (0, 0)))

    w2p2 = mlp2_w2.reshape(DIM * DIM, DIM)
    b2p2 = mlp2_b2.reshape(DIM, DIM)

    def layer(xtab, w1, b1, w2p, b2p, rootp, bias, inp):
        rmat = jnp.repeat(jnp.eye(DIM, dtype=jnp.float32), inp, axis=1)
        smat = jnp.tile(jnp.eye(inp, dtype=jnp.float32), (1, DIM))
        xs = _sc_gather(xtab, src2d, inp)
        msg = _tc_edge(ea_p, xs, w1, b1.reshape(1, DIM), w2p, b2p,
                       rmat, smat, inp)
        agg = _sc_scatter(msg, dst2d)
        return _tc_node(agg, xtab, rootp, bias.reshape(1, DIM), inp)

    x1 = layer(xpad, mlp1_w1, mlp1_b1, w2p1, b2p1, root1p, bias1, 16)
    x2 = layer(x1, mlp2_w1, mlp2_b1, w2p2, b2p2, root2, bias2, DIM)
    return x2


# speedup vs baseline: 2.7669x; 1.0187x over previous
import functools

import jax
import jax.numpy as jnp
from jax import lax
from jax.experimental import pallas as pl
from jax.experimental.pallas import tpu as pltpu
from jax.experimental.pallas import tpu_sc as plsc

N = 100000
E = 1600000
DIM = 32
NF = 11
EF = 4

NC = 2
NS = 16
NW = NC * NS

E_PAD = 1638400
CH = 1024
EPW = E_PAD // NW
EPS = E_PAD // NS
NQ = N // 4
SH = NQ + 8
TRASH = NQ

_mesh = plsc.VectorSubcoreMesh(core_axis_name="c", subcore_axis_name="s",
                               num_cores=NC, num_subcores=NS)


def _sc_gather(table, idx2d, F):
    @functools.partial(
        pl.kernel, mesh=_mesh,
        out_type=jax.ShapeDtypeStruct((E_PAD, F), jnp.float32),
        compiler_params=pltpu.CompilerParams(use_tc_tiling_on_sc=False),
        scratch_types=[
            pltpu.VMEM((2, 8, 128), jnp.int32),
            pltpu.VMEM((2, CH, F), jnp.float32),
            pltpu.SemaphoreType.DMA,
            pltpu.SemaphoreType.DMA,
            pltpu.SemaphoreType.DMA,
            pltpu.SemaphoreType.DMA,
            pltpu.SemaphoreType.DMA,
        ],
    )
    def k(table_hbm, idx_hbm, out_hbm, idx_v, rows_v, gsem,
          isem0, isem1, wsem0, wsem1):
        isem = [isem0, isem1]
        wsem = [wsem0, wsem1]
        wid = lax.axis_index("s") * NC + lax.axis_index("c")
        nch = EPW // CH

        def start_idx(ci, b):
            e0 = wid * EPW + ci * CH
            r0 = pl.multiple_of(e0 // 128, 8)
            pltpu.async_copy(idx_hbm.at[pl.ds(r0, 8)], idx_v.at[b], isem[b])

        def drain_idx(b):
            pltpu.make_async_copy(idx_hbm.at[pl.ds(0, 8)],
                                  idx_v.at[b], isem[b]).wait()

        def drain_out(b):
            pltpu.make_async_copy(rows_v.at[b],
                                  out_hbm.at[pl.ds(0, CH)], wsem[b]).wait()

        start_idx(0, 0)
        start_idx(1, 1)

        def body(g, _):
            for b in range(2):
                ci = g * 2 + b
                e0 = wid * EPW + ci * CH
                drain_idx(b)
                @pl.when(ci >= 2)
                def _():
                    drain_out(b)
                descs = []
                for j in range(8):
                    descs.append(pltpu.async_copy(
                        table_hbm.at[idx_v.at[b, j]],
                        rows_v.at[b, pl.ds(j * 128, 128)], gsem))
                for d in descs:
                    d.wait()
                pltpu.async_copy(rows_v.at[b], out_hbm.at[pl.ds(e0, CH)],
                                 wsem[b])

                @pl.when(ci + 2 < nch)
                def _():
                    start_idx(ci + 2, b)
            return 0

        lax.fori_loop(0, nch // 2, body, 0)
        drain_out(0)
        drain_out(1)

    return k(table, idx2d)


def _sc_scatter(msg, dst2d):
    @functools.partial(
        pl.kernel, mesh=_mesh,
        out_type=jax.ShapeDtypeStruct((N, DIM), jnp.float32),
        compiler_params=pltpu.CompilerParams(use_tc_tiling_on_sc=False),
        scratch_types=[
            pltpu.VMEM((2, 8, 128), jnp.int32),
            pltpu.VMEM((8, 128), jnp.int32),
            pltpu.VMEM((2, CH, DIM), jnp.float32),
            pltpu.VMEM_SHARED((SH, DIM), jnp.float32),
            pltpu.SemaphoreType.DMA,
            pltpu.SemaphoreType.DMA,
            pltpu.SemaphoreType.DMA,
            pltpu.SemaphoreType.DMA,
            pltpu.SemaphoreType.DMA,
        ],
    )
    def k(msg_hbm, dst_hbm, out_hbm, raw_v, idx_v, val_v, shared,
          ssem, rsem0, rsem1, msem0, msem1):
        c = lax.axis_index("c")
        s = lax.axis_index("s")
        rsem = [rsem0, rsem1]
        msem = [msem0, msem1]
        nchp = EPS // CH

        zer = jnp.zeros((16,), jnp.float32)

        def zfill(i, _):
            val_v[0, i, pl.ds(0, 16)] = zer
            val_v[0, i, pl.ds(16, 16)] = zer
            return 0

        def start_loads(ci, b):
            e0 = s * EPS + ci * CH
            r0 = pl.multiple_of(e0 // 128, 8)
            pltpu.async_copy(dst_hbm.at[pl.ds(r0, 8)], raw_v.at[b], rsem[b])
            pltpu.async_copy(msg_hbm.at[pl.ds(e0, CH)], val_v.at[b], msem[b])

        def drain_loads(b):
            pltpu.make_async_copy(dst_hbm.at[pl.ds(0, 8)],
                                  raw_v.at[b], rsem[b]).wait()
            pltpu.make_async_copy(msg_hbm.at[pl.ds(0, CH)],
                                  val_v.at[b], msem[b]).wait()

        for p in range(2):
            lax.fori_loop(0, CH, zfill, 0)
            z0 = s * (SH // NS)
            pltpu.sync_copy(val_v.at[0], shared.at[pl.ds(z0, CH)])
            pltpu.sync_copy(val_v.at[0, pl.ds(0, SH // NS - CH)],
                            shared.at[pl.ds(z0 + CH, SH // NS - CH)])
            plsc.subcore_barrier()

            lo = (2 * p + c) * NQ
            start_loads(0, 0)
            start_loads(1, 1)

            def body2(g, _):
                for b in range(2):
                    ci = g * 2 + b
                    drain_loads(b)
                    for j in range(8):
                        for m in range(8):
                            v = raw_v[b, j, pl.ds(m * 16, 16)]
                            rel = v - lo
                            ok = (rel >= 0) & (rel < NQ)
                            idx_v[j, pl.ds(m * 16, 16)] = jnp.where(
                                ok, rel, TRASH)
                    descs = []
                    for j in range(8):
                        descs.append(pltpu.async_copy(
                            val_v.at[b, pl.ds(j * 128, 128)],
                            shared.at[idx_v.at[j]], ssem, add=True))
                    for d in descs:
                        d.wait()

                    @pl.when(ci + 2 < nchp)
                    def _():
                        start_loads(ci + 2, b)
                return 0

            lax.fori_loop(0, nchp // 2, body2, 0)
            plsc.subcore_barrier()

            rows = NQ // NS
            pltpu.sync_copy(shared.at[pl.ds(s * rows, rows)],
                            out_hbm.at[pl.ds(lo + s * rows, rows)])

            @pl.when(s == 0)
            def _():
                pltpu.sync_copy(shared.at[pl.ds(NS * rows, NQ - NS * rows)],
                                out_hbm.at[pl.ds(lo + NS * rows, NQ - NS * rows)])

            plsc.subcore_barrier()

    return k(msg, dst2d)


def _tc_edge(ea_t, xs, w1, b1, w2p, b2p, rmat, smat, inp):
    T = 512
    G = E // T

    def body(ea_ref, xs_ref, w1_ref, b1_ref, w2p_ref, b2p_ref, r_ref, s_ref,
             out_ref):
        ea = ea_ref[...]
        xsv = xs_ref[...]
        h = jnp.maximum(
            jax.lax.dot_general(ea, w1_ref[...], (((0,), (0,)), ((), ())),
                                preferred_element_type=jnp.float32)
            + b1_ref[...], 0.0)
        hq = jnp.dot(h, r_ref[...], preferred_element_type=jnp.float32)
        xq = jnp.dot(xsv, s_ref[...], preferred_element_type=jnp.float32)
        q = hq * xq
        out_ref[...] = (
            jnp.dot(q, w2p_ref[...], preferred_element_type=jnp.float32)
            + jnp.dot(xsv, b2p_ref[...], preferred_element_type=jnp.float32))

    return pl.pallas_call(
        body,
        grid=(G,),
        in_specs=[
            pl.BlockSpec((EF, T), lambda i: (0, i)),
            pl.BlockSpec((T, inp), lambda i: (i, 0)),
            pl.BlockSpec((EF, DIM), lambda i: (0, 0)),
            pl.BlockSpec((1, DIM), lambda i: (0, 0)),
            pl.BlockSpec((DIM * inp, DIM), lambda i: (0, 0)),
            pl.BlockSpec((inp, DIM), lambda i: (0, 0)),
            pl.BlockSpec((DIM, DIM * inp), lambda i: (0, 0)),
            pl.BlockSpec((inp, DIM * inp), lambda i: (0, 0)),
        ],
        out_specs=pl.BlockSpec((T, DIM), lambda i: (i, 0)),
        out_shape=jax.ShapeDtypeStruct((E_PAD, DIM), jnp.float32),
    )(ea_t, xs, w1, b1, w2p, b2p, rmat, smat)


def _tc_node(agg, xtab, rootp, bias, inp):
    T = 1000
    G = N // T

    def body(agg_ref, x_ref, r_ref, b_ref, out_ref):
        out_ref[...] = jnp.maximum(
            agg_ref[...]
            + jnp.dot(x_ref[...], r_ref[...], preferred_element_type=jnp.float32)
            + b_ref[...], 0.0)

    return pl.pallas_call(
        body,
        grid=(G,),
        in_specs=[
            pl.BlockSpec((T, DIM), lambda i: (i, 0)),
            pl.BlockSpec((T, inp), lambda i: (i, 0)),
            pl.BlockSpec((inp, DIM), lambda i: (0, 0)),
            pl.BlockSpec((1, DIM), lambda i: (0, 0)),
        ],
        out_specs=pl.BlockSpec((T, DIM), lambda i: (i, 0)),
        out_shape=jax.ShapeDtypeStruct((N, DIM), jnp.float32),
    )(agg, xtab, rootp, bias)


def kernel(x, edge_index, edge_attr, batch,
           mlp1_w1, mlp1_b1, mlp1_w2, mlp1_b2, root1, bias1,
           mlp2_w1, mlp2_b1, mlp2_w2, mlp2_b2, root2, bias2):
    del batch
    src = edge_index[0].astype(jnp.int32)
    dst = edge_index[1].astype(jnp.int32)
    src2d = jnp.pad(src, (0, E_PAD - E)).reshape(E_PAD // 128, 128)
    dst2d = jnp.pad(dst, (0, E_PAD - E),
                    constant_values=N).reshape(E_PAD // 128, 128)
    ea_t = edge_attr.T

    xpad = jnp.pad(x, ((0, 0), (0, 16 - NF)))
    w2r1 = mlp1_w2.reshape(DIM, NF, DIM)
    w2p1 = jnp.pad(w2r1, ((0, 0), (0, 16 - NF), (0, 0))).reshape(DIM * 16, DIM)
    b2p1 = jnp.pad(mlp1_b2.reshape(NF, DIM), ((0, 16 - NF), (0, 0)))
    root1p = jnp.pad(root1, ((0, 16 - NF), (0, 0)))

    w2p2 = mlp2_w2.reshape(DIM * DIM, DIM)
    b2p2 = mlp2_b2.reshape(DIM, DIM)

    def layer(xtab, w1, b1, w2p, b2p, rootp, bias, inp):
        rmat = jnp.repeat(jnp.eye(DIM, dtype=jnp.float32), inp, axis=1)
        smat = jnp.tile(jnp.eye(inp, dtype=jnp.float32), (1, DIM))
        xs = _sc_gather(xtab, src2d, inp)
        msg = _tc_edge(ea_t, xs, w1, b1.reshape(1, DIM), w2p, b2p,
                       rmat, smat, inp)
        agg = _sc_scatter(msg, dst2d)
        return _tc_node(agg, xtab, rootp, bias.reshape(1, DIM), inp)

    x1 = layer(xpad, mlp1_w1, mlp1_b1, w2p1, b2p1, root1p, bias1, 16)
    x2 = layer(x1, mlp2_w1, mlp2_b1, w2p2, b2p2, root2, bias2, DIM)
    return x2


# speedup vs baseline: 3.3460x; 1.2093x over previous
import functools

import jax
import jax.numpy as jnp
from jax import lax
from jax.experimental import pallas as pl
from jax.experimental.pallas import tpu as pltpu
from jax.experimental.pallas import tpu_sc as plsc

N = 100000
E = 1600000
DIM = 32
NF = 11
EF = 4

NC = 2
NS = 16
NW = NC * NS

E_PAD = 1638400
CH = 1024
EPW = E_PAD // NW
EPS = E_PAD // NS
NH = N // 2
SH = NH + 16
TRASH = NH
HD = DIM // 2

_mesh = plsc.VectorSubcoreMesh(core_axis_name="c", subcore_axis_name="s",
                               num_cores=NC, num_subcores=NS)


def _sc_gather(table, idx2d, F):
    @functools.partial(
        pl.kernel, mesh=_mesh,
        out_type=jax.ShapeDtypeStruct((E_PAD, F), jnp.float32),
        compiler_params=pltpu.CompilerParams(use_tc_tiling_on_sc=False),
        scratch_types=[
            pltpu.VMEM((2, 8, 128), jnp.int32),
            pltpu.VMEM((2, CH, F), jnp.float32),
            pltpu.SemaphoreType.DMA,
            pltpu.SemaphoreType.DMA,
            pltpu.SemaphoreType.DMA,
            pltpu.SemaphoreType.DMA,
            pltpu.SemaphoreType.DMA,
        ],
    )
    def k(table_hbm, idx_hbm, out_hbm, idx_v, rows_v, gsem,
          isem0, isem1, wsem0, wsem1):
        isem = [isem0, isem1]
        wsem = [wsem0, wsem1]
        wid = lax.axis_index("s") * NC + lax.axis_index("c")
        nch = EPW // CH

        def start_idx(ci, b):
            e0 = wid * EPW + ci * CH
            r0 = pl.multiple_of(e0 // 128, 8)
            pltpu.async_copy(idx_hbm.at[pl.ds(r0, 8)], idx_v.at[b], isem[b])

        def drain_idx(b):
            pltpu.make_async_copy(idx_hbm.at[pl.ds(0, 8)],
                                  idx_v.at[b], isem[b]).wait()

        def drain_out(b):
            pltpu.make_async_copy(rows_v.at[b],
                                  out_hbm.at[pl.ds(0, CH)], wsem[b]).wait()

        start_idx(0, 0)
        start_idx(1, 1)

        def body(g, _):
            for b in range(2):
                ci = g * 2 + b
                e0 = wid * EPW + ci * CH
                drain_idx(b)
                @pl.when(ci >= 2)
                def _():
                    drain_out(b)
                descs = []
                for j in range(8):
                    descs.append(pltpu.async_copy(
                        table_hbm.at[idx_v.at[b, j]],
                        rows_v.at[b, pl.ds(j * 128, 128)], gsem))
                for d in descs:
                    d.wait()
                pltpu.async_copy(rows_v.at[b], out_hbm.at[pl.ds(e0, CH)],
                                 wsem[b])

                @pl.when(ci + 2 < nch)
                def _():
                    start_idx(ci + 2, b)
            return 0

        lax.fori_loop(0, nch // 2, body, 0)
        drain_out(0)
        drain_out(1)

    return k(table, idx2d)


def _sc_scatter(msg128, dst2d):
    @functools.partial(
        pl.kernel, mesh=_mesh,
        out_type=jax.ShapeDtypeStruct((N, DIM), jnp.float32),
        compiler_params=pltpu.CompilerParams(use_tc_tiling_on_sc=False),
        scratch_types=[
            pltpu.VMEM((2, 8, 128), jnp.int32),
            pltpu.VMEM((8, 128), jnp.int32),
            pltpu.VMEM((2, CH, HD), jnp.float32),
            pltpu.VMEM_SHARED((SH, HD), jnp.float32),
            pltpu.SemaphoreType.DMA,
            pltpu.SemaphoreType.DMA,
            pltpu.SemaphoreType.DMA,
            pltpu.SemaphoreType.DMA,
            pltpu.SemaphoreType.DMA,
        ],
    )
    def k(msg_hbm, dst_hbm, out_hbm, raw_v, idx_v, val_v, shared,
          ssem, rsem0, rsem1, msem0, msem1):
        c = lax.axis_index("c")
        s = lax.axis_index("s")
        rsem = [rsem0, rsem1]
        msem = [msem0, msem1]
        nchp = EPS // CH
        col0 = c * HD

        zer = jnp.zeros((16,), jnp.float32)

        def zfill(i, _):
            val_v[0, i, pl.ds(0, 16)] = zer
            return 0

        def start_loads(ci, b):
            e0 = s * EPS + ci * CH
            r0 = pl.multiple_of(e0 // 128, 8)
            pltpu.async_copy(dst_hbm.at[pl.ds(r0, 8)], raw_v.at[b], rsem[b])
            pltpu.async_copy(msg_hbm.at[pl.ds(e0, CH), pl.ds(col0, HD)],
                             val_v.at[b], msem[b])

        def drain_loads(b):
            pltpu.make_async_copy(dst_hbm.at[pl.ds(0, 8)],
                                  raw_v.at[b], rsem[b]).wait()
            pltpu.make_async_copy(msg_hbm.at[pl.ds(0, CH), pl.ds(col0, HD)],
                                  val_v.at[b], msem[b]).wait()

        for p in range(2):
            lax.fori_loop(0, CH, zfill, 0)
            z0 = s * (SH // NS)
            done = 0
            while done < SH // NS:
                sz = min(CH, SH // NS - done)
                pltpu.sync_copy(val_v.at[0, pl.ds(0, sz)],
                                shared.at[pl.ds(z0 + done, sz)])
                done += sz
            plsc.subcore_barrier()

            lo = p * NH
            start_loads(0, 0)
            start_loads(1, 1)

            def body2(g, _):
                for b in range(2):
                    ci = g * 2 + b
                    drain_loads(b)
                    for j in range(8):
                        for m in range(8):
                            v = raw_v[b, j, pl.ds(m * 16, 16)]
                            rel = v - lo
                            ok = (rel >= 0) & (rel < NH)
                            idx_v[j, pl.ds(m * 16, 16)] = jnp.where(
                                ok, rel, TRASH)
                    descs = []
                    for j in range(8):
                        descs.append(pltpu.async_copy(
                            val_v.at[b, pl.ds(j * 128, 128)],
                            shared.at[idx_v.at[j]], ssem, add=True))
                    for d in descs:
                        d.wait()

                    @pl.when(ci + 2 < nchp)
                    def _():
                        start_loads(ci + 2, b)
                return 0

            lax.fori_loop(0, nchp // 2, body2, 0)
            plsc.subcore_barrier()

            rows = NH // NS
            pltpu.sync_copy(
                shared.at[pl.ds(s * rows, rows)],
                out_hbm.at[pl.ds(lo + s * rows, rows), pl.ds(col0, HD)])
            plsc.subcore_barrier()

    return k(msg128, dst2d)


def _tc_edge(ea_t, xs, w1, b1, w2p, b2p, rmat, smat, inp):
    T = 512
    G = E // T

    def body(ea_ref, xs_ref, w1_ref, b1_ref, w2p_ref, b2p_ref, r_ref, s_ref,
             out_ref):
        ea = ea_ref[...]
        xsv = xs_ref[...]
        h = jnp.maximum(
            jax.lax.dot_general(ea, w1_ref[...], (((0,), (0,)), ((), ())),
                                preferred_element_type=jnp.float32)
            + b1_ref[...], 0.0)
        hq = jnp.dot(h, r_ref[...], preferred_element_type=jnp.float32)
        xq = jnp.dot(xsv, s_ref[...], preferred_element_type=jnp.float32)
        q = hq * xq
        out_ref[:, :DIM] = (
            jnp.dot(q, w2p_ref[...], preferred_element_type=jnp.float32)
            + jnp.dot(xsv, b2p_ref[...], preferred_element_type=jnp.float32))

    return pl.pallas_call(
        body,
        grid=(G,),
        in_specs=[
            pl.BlockSpec((EF, T), lambda i: (0, i)),
            pl.BlockSpec((T, inp), lambda i: (i, 0)),
            pl.BlockSpec((EF, DIM), lambda i: (0, 0)),
            pl.BlockSpec((1, DIM), lambda i: (0, 0)),
            pl.BlockSpec((DIM * inp, DIM), lambda i: (0, 0)),
            pl.BlockSpec((inp, DIM), lambda i: (0, 0)),
            pl.BlockSpec((DIM, DIM * inp), lambda i: (0, 0)),
            pl.BlockSpec((inp, DIM * inp), lambda i: (0, 0)),
        ],
        out_specs=pl.BlockSpec((T, 128), lambda i: (i, 0)),
        out_shape=jax.ShapeDtypeStruct((E_PAD, 128), jnp.float32),
    )(ea_t, xs, w1, b1, w2p, b2p, rmat, smat)


def _tc_node(agg, xtab, rootp, bias, inp):
    T = 1000
    G = N // T

    def body(agg_ref, x_ref, r_ref, b_ref, out_ref):
        out_ref[...] = jnp.maximum(
            agg_ref[...]
            + jnp.dot(x_ref[...], r_ref[...], preferred_element_type=jnp.float32)
            + b_ref[...], 0.0)

    return pl.pallas_call(
        body,
        grid=(G,),
        in_specs=[
            pl.BlockSpec((T, DIM), lambda i: (i, 0)),
            pl.BlockSpec((T, inp), lambda i: (i, 0)),
            pl.BlockSpec((inp, DIM), lambda i: (0, 0)),
            pl.BlockSpec((1, DIM), lambda i: (0, 0)),
        ],
        out_specs=pl.BlockSpec((T, DIM), lambda i: (i, 0)),
        out_shape=jax.ShapeDtypeStruct((N, DIM), jnp.float32),
    )(agg, xtab, rootp, bias)


def kernel(x, edge_index, edge_attr, batch,
           mlp1_w1, mlp1_b1, mlp1_w2, mlp1_b2, root1, bias1,
           mlp2_w1, mlp2_b1, mlp2_w2, mlp2_b2, root2, bias2):
    del batch
    src = edge_index[0].astype(jnp.int32)
    dst = edge_index[1].astype(jnp.int32)
    src2d = jnp.pad(src, (0, E_PAD - E)).reshape(E_PAD // 128, 128)
    dst2d = jnp.pad(dst, (0, E_PAD - E),
                    constant_values=N).reshape(E_PAD // 128, 128)
    ea_t = edge_attr.T

    xpad = jnp.pad(x, ((0, 0), (0, 16 - NF)))
    w2r1 = mlp1_w2.reshape(DIM, NF, DIM)
    w2p1 = jnp.pad(w2r1, ((0, 0), (0, 16 - NF), (0, 0))).reshape(DIM * 16, DIM)
    b2p1 = jnp.pad(mlp1_b2.reshape(NF, DIM), ((0, 16 - NF), (0, 0)))
    root1p = jnp.pad(root1, ((0, 16 - NF), (0, 0)))

    w2p2 = mlp2_w2.reshape(DIM * DIM, DIM)
    b2p2 = mlp2_b2.reshape(DIM, DIM)

    def layer(xtab, w1, b1, w2p, b2p, rootp, bias, inp):
        rmat = jnp.repeat(jnp.eye(DIM, dtype=jnp.float32), inp, axis=1)
        smat = jnp.tile(jnp.eye(inp, dtype=jnp.float32), (1, DIM))
        xs = _sc_gather(xtab, src2d, inp)
        msg = _tc_edge(ea_t, xs, w1, b1.reshape(1, DIM), w2p, b2p,
                       rmat, smat, inp)
        agg = _sc_scatter(msg, dst2d)
        return _tc_node(agg, xtab, rootp, bias.reshape(1, DIM), inp)

    x1 = layer(xpad, mlp1_w1, mlp1_b1, w2p1, b2p1, root1p, bias1, 16)
    x2 = layer(x1, mlp2_w1, mlp2_b1, w2p2, b2p2, root2, bias2, DIM)
    return x2


# speedup vs baseline: 3.7962x; 1.1345x over previous
import functools

import jax
import jax.numpy as jnp
from jax import lax
from jax.experimental import pallas as pl
from jax.experimental.pallas import tpu as pltpu
from jax.experimental.pallas import tpu_sc as plsc

N = 100000
E = 1600000
DIM = 32
NF = 11
EF = 4

NC = 2
NS = 16
NW = NC * NS

E_PAD = 1638400
CH = 1024
EPW = E_PAD // NW
EPS = E_PAD // NS
NH = N // 2
SH = NH + 16
TRASH = NH
HD = DIM // 2

_mesh = plsc.VectorSubcoreMesh(core_axis_name="c", subcore_axis_name="s",
                               num_cores=NC, num_subcores=NS)


def _sc_gather(table, idx2d, F):
    @functools.partial(
        pl.kernel, mesh=_mesh,
        out_type=jax.ShapeDtypeStruct((E_PAD, F), jnp.float32),
        compiler_params=pltpu.CompilerParams(use_tc_tiling_on_sc=False),
        scratch_types=[
            pltpu.VMEM((2, 8, 128), jnp.int32),
            pltpu.VMEM((2, CH, F), jnp.float32),
            pltpu.SemaphoreType.DMA,
            pltpu.SemaphoreType.DMA,
            pltpu.SemaphoreType.DMA,
            pltpu.SemaphoreType.DMA,
            pltpu.SemaphoreType.DMA,
        ],
    )
    def k(table_hbm, idx_hbm, out_hbm, idx_v, rows_v, gsem,
          isem0, isem1, wsem0, wsem1):
        isem = [isem0, isem1]
        wsem = [wsem0, wsem1]
        wid = lax.axis_index("s") * NC + lax.axis_index("c")
        nch = EPW // CH

        def start_idx(ci, b):
            e0 = wid * EPW + ci * CH
            r0 = pl.multiple_of(e0 // 128, 8)
            pltpu.async_copy(idx_hbm.at[pl.ds(r0, 8)], idx_v.at[b], isem[b])

        def drain_idx(b):
            pltpu.make_async_copy(idx_hbm.at[pl.ds(0, 8)],
                                  idx_v.at[b], isem[b]).wait()

        def drain_out(b):
            pltpu.make_async_copy(rows_v.at[b],
                                  out_hbm.at[pl.ds(0, CH)], wsem[b]).wait()

        start_idx(0, 0)
        start_idx(1, 1)

        def body(g, _):
            for b in range(2):
                ci = g * 2 + b
                e0 = wid * EPW + ci * CH
                drain_idx(b)
                @pl.when(ci >= 2)
                def _():
                    drain_out(b)
                descs = []
                for j in range(8):
                    descs.append(pltpu.async_copy(
                        table_hbm.at[idx_v.at[b, j]],
                        rows_v.at[b, pl.ds(j * 128, 128)], gsem))
                for d in descs:
                    d.wait()
                pltpu.async_copy(rows_v.at[b], out_hbm.at[pl.ds(e0, CH)],
                                 wsem[b])

                @pl.when(ci + 2 < nch)
                def _():
                    start_idx(ci + 2, b)
            return 0

        lax.fori_loop(0, nch // 2, body, 0)
        drain_out(0)
        drain_out(1)

    return k(table, idx2d)


def _sc_scatter(msg128, dst2d):
    @functools.partial(
        pl.kernel, mesh=_mesh,
        out_type=jax.ShapeDtypeStruct((N, DIM), jnp.float32),
        compiler_params=pltpu.CompilerParams(use_tc_tiling_on_sc=False),
        scratch_types=[
            pltpu.VMEM((2, 8, 128), jnp.int32),
            pltpu.VMEM((8, 128), jnp.int32),
            pltpu.VMEM((2, CH, HD), jnp.float32),
            pltpu.VMEM_SHARED((SH, HD), jnp.float32),
            pltpu.SemaphoreType.DMA,
            pltpu.SemaphoreType.DMA,
            pltpu.SemaphoreType.DMA,
            pltpu.SemaphoreType.DMA,
            pltpu.SemaphoreType.DMA,
        ],
    )
    def k(msg_hbm, dst_hbm, out_hbm, raw_v, idx_v, val_v, shared,
          ssem, rsem0, rsem1, msem0, msem1):
        c = lax.axis_index("c")
        s = lax.axis_index("s")
        rsem = [rsem0, rsem1]
        msem = [msem0, msem1]
        nchp = EPS // CH
        col0 = c * HD

        zer = jnp.zeros((16,), jnp.float32)

        def zfill(i, _):
            val_v[0, i, pl.ds(0, 16)] = zer
            return 0

        def start_loads(ci, b):
            e0 = s * EPS + ci * CH
            r0 = pl.multiple_of(e0 // 128, 8)
            pltpu.async_copy(dst_hbm.at[pl.ds(r0, 8)], raw_v.at[b], rsem[b])
            pltpu.async_copy(msg_hbm.at[pl.ds(e0, CH), pl.ds(col0, HD)],
                             val_v.at[b], msem[b])

        def drain_loads(b):
            pltpu.make_async_copy(dst_hbm.at[pl.ds(0, 8)],
                                  raw_v.at[b], rsem[b]).wait()
            pltpu.make_async_copy(msg_hbm.at[pl.ds(0, CH), pl.ds(col0, HD)],
                                  val_v.at[b], msem[b]).wait()

        for p in range(2):
            lax.fori_loop(0, CH, zfill, 0)
            z0 = s * (SH // NS)
            done = 0
            while done < SH // NS:
                sz = min(CH, SH // NS - done)
                pltpu.sync_copy(val_v.at[0, pl.ds(0, sz)],
                                shared.at[pl.ds(z0 + done, sz)])
                done += sz
            plsc.subcore_barrier()

            lo = p * NH
            start_loads(0, 0)
            start_loads(1, 1)

            def body2(g, _):
                for b in range(2):
                    ci = g * 2 + b
                    drain_loads(b)
                    for j in range(8):
                        for m in range(8):
                            v = raw_v[b, j, pl.ds(m * 16, 16)]
                            rel = v - lo
                            ok = (rel >= 0) & (rel < NH)
                            idx_v[j, pl.ds(m * 16, 16)] = jnp.where(
                                ok, rel, TRASH)
                    descs = []
                    for j in range(8):
                        descs.append(pltpu.async_copy(
                            val_v.at[b, pl.ds(j * 128, 128)],
                            shared.at[idx_v.at[j]], ssem, add=True))
                    for d in descs:
                        d.wait()

                    @pl.when(ci + 2 < nchp)
                    def _():
                        start_loads(ci + 2, b)
                return 0

            lax.fori_loop(0, nchp // 2, body2, 0)
            plsc.subcore_barrier()

            rows = NH // NS
            pltpu.sync_copy(
                shared.at[pl.ds(s * rows, rows)],
                out_hbm.at[pl.ds(lo + s * rows, rows), pl.ds(col0, HD)])
            plsc.subcore_barrier()

    return k(msg128, dst2d)


def _tc_edge(ea_t, xs, w1, b1, w2p, b2p, rmat, smat, inp):
    T = 1024
    G = (E + T - 1) // T

    def body(ea_ref, xs_ref, w1_ref, b1_ref, w2p_ref, b2p_ref, r_ref, s_ref,
             out_ref):
        ea = ea_ref[...]
        xsv = xs_ref[...]
        h = jnp.maximum(
            jax.lax.dot_general(ea, w1_ref[...], (((0,), (0,)), ((), ())),
                                preferred_element_type=jnp.float32)
            + b1_ref[...], 0.0)
        hq = jnp.dot(h, r_ref[...], preferred_element_type=jnp.float32)
        xq = jnp.dot(xsv, s_ref[...], preferred_element_type=jnp.float32)
        q = hq * xq
        out_ref[:, :DIM] = (
            jnp.dot(q, w2p_ref[...], preferred_element_type=jnp.float32)
            + jnp.dot(xsv, b2p_ref[...], preferred_element_type=jnp.float32))

    return pl.pallas_call(
        body,
        grid=(G,),
        in_specs=[
            pl.BlockSpec((EF, T), lambda i: (0, i)),
            pl.BlockSpec((T, inp), lambda i: (i, 0)),
            pl.BlockSpec((EF, DIM), lambda i: (0, 0)),
            pl.BlockSpec((1, DIM), lambda i: (0, 0)),
            pl.BlockSpec((DIM * inp, DIM), lambda i: (0, 0)),
            pl.BlockSpec((inp, DIM), lambda i: (0, 0)),
            pl.BlockSpec((DIM, DIM * inp), lambda i: (0, 0)),
            pl.BlockSpec((inp, DIM * inp), lambda i: (0, 0)),
        ],
        out_specs=pl.BlockSpec((T, 128), lambda i: (i, 0)),
        out_shape=jax.ShapeDtypeStruct((E_PAD, 128), jnp.float32),
    )(ea_t, xs, w1, b1, w2p, b2p, rmat, smat)


def _tc_node(agg, xtab, rootp, bias, inp):
    T = 1000
    G = N // T

    def body(agg_ref, x_ref, r_ref, b_ref, out_ref):
        out_ref[...] = jnp.maximum(
            agg_ref[...]
            + jnp.dot(x_ref[...], r_ref[...], preferred_element_type=jnp.float32)
            + b_ref[...], 0.0)

    return pl.pallas_call(
        body,
        grid=(G,),
        in_specs=[
            pl.BlockSpec((T, DIM), lambda i: (i, 0)),
            pl.BlockSpec((T, inp), lambda i: (i, 0)),
            pl.BlockSpec((inp, DIM), lambda i: (0, 0)),
            pl.BlockSpec((1, DIM), lambda i: (0, 0)),
        ],
        out_specs=pl.BlockSpec((T, DIM), lambda i: (i, 0)),
        out_shape=jax.ShapeDtypeStruct((N, DIM), jnp.float32),
    )(agg, xtab, rootp, bias)


def kernel(x, edge_index, edge_attr, batch,
           mlp1_w1, mlp1_b1, mlp1_w2, mlp1_b2, root1, bias1,
           mlp2_w1, mlp2_b1, mlp2_w2, mlp2_b2, root2, bias2):
    del batch
    src = edge_index[0].astype(jnp.int32)
    dst = edge_index[1].astype(jnp.int32)
    src2d = jnp.pad(src, (0, E_PAD - E)).reshape(E_PAD // 128, 128)
    dst2d = jnp.pad(dst, (0, E_PAD - E),
                    constant_values=N).reshape(E_PAD // 128, 128)
    ea_t = jnp.pad(edge_attr.T, ((0, 0), (0, 1024 * 1563 - E)))

    xpad = jnp.pad(x, ((0, 0), (0, 16 - NF)))
    w2r1 = mlp1_w2.reshape(DIM, NF, DIM)
    w2p1 = jnp.pad(w2r1, ((0, 0), (0, 16 - NF), (0, 0))).reshape(DIM * 16, DIM)
    b2p1 = jnp.pad(mlp1_b2.reshape(NF, DIM), ((0, 16 - NF), (0, 0)))
    root1p = jnp.pad(root1, ((0, 16 - NF), (0, 0)))

    w2p2 = mlp2_w2.reshape(DIM * DIM, DIM)
    b2p2 = mlp2_b2.reshape(DIM, DIM)

    def layer(xtab, w1, b1, w2p, b2p, rootp, bias, inp):
        rmat = jnp.repeat(jnp.eye(DIM, dtype=jnp.float32), inp, axis=1)
        smat = jnp.tile(jnp.eye(inp, dtype=jnp.float32), (1, DIM))
        xs = _sc_gather(xtab, src2d, inp)
        msg = _tc_edge(ea_t, xs, w1, b1.reshape(1, DIM), w2p, b2p,
                       rmat, smat, inp)
        agg = _sc_scatter(msg, dst2d)
        return _tc_node(agg, xtab, rootp, bias.reshape(1, DIM), inp)

    x1 = layer(xpad, mlp1_w1, mlp1_b1, w2p1, b2p1, root1p, bias1, 16)
    x2 = layer(x1, mlp2_w1, mlp2_b1, w2p2, b2p2, root2, bias2, DIM)
    return x2


# speedup vs baseline: 4.6147x; 1.2156x over previous
import functools

import jax
import jax.numpy as jnp
from jax import lax
from jax.experimental import pallas as pl
from jax.experimental.pallas import tpu as pltpu
from jax.experimental.pallas import tpu_sc as plsc

N = 100000
E = 1600000
DIM = 32
NF = 11
EF = 4

NC = 2
NS = 16
NW = NC * NS

E_PAD = 1638400
EH = E_PAD // 2
CH = 1024
EPW = EH // NW
EPS = EH // NS
NH = N // 2
SH = NH + 16
TRASH = NH
HD = DIM // 2

_mesh = plsc.VectorSubcoreMesh(core_axis_name="c", subcore_axis_name="s",
                               num_cores=NC, num_subcores=NS)


def _sc_gather(table, idx2d, F):
    @functools.partial(
        pl.kernel, mesh=_mesh,
        out_type=jax.ShapeDtypeStruct((EH, F), jnp.float32),
        compiler_params=pltpu.CompilerParams(use_tc_tiling_on_sc=False),
        scratch_types=[
            pltpu.VMEM((2, 8, 128), jnp.int32),
            pltpu.VMEM((2, CH, F), jnp.float32),
            pltpu.SemaphoreType.DMA,
            pltpu.SemaphoreType.DMA,
            pltpu.SemaphoreType.DMA,
            pltpu.SemaphoreType.DMA,
            pltpu.SemaphoreType.DMA,
        ],
    )
    def k(table_hbm, idx_hbm, out_hbm, idx_v, rows_v, gsem,
          isem0, isem1, wsem0, wsem1):
        isem = [isem0, isem1]
        wsem = [wsem0, wsem1]
        wid = lax.axis_index("s") * NC + lax.axis_index("c")
        nch = EPW // CH

        def start_idx(ci, b):
            e0 = wid * EPW + ci * CH
            r0 = pl.multiple_of(e0 // 128, 8)
            pltpu.async_copy(idx_hbm.at[pl.ds(r0, 8)], idx_v.at[b], isem[b])

        def drain_idx(b):
            pltpu.make_async_copy(idx_hbm.at[pl.ds(0, 8)],
                                  idx_v.at[b], isem[b]).wait()

        def drain_out(b):
            pltpu.make_async_copy(rows_v.at[b],
                                  out_hbm.at[pl.ds(0, CH)], wsem[b]).wait()

        def process(ci, b, tail):
            e0 = wid * EPW + ci * CH
            drain_idx(b)
            if tail:
                drain_out(b)
            else:
                @pl.when(ci >= 2)
                def _():
                    drain_out(b)
            descs = []
            for j in range(8):
                descs.append(pltpu.async_copy(
                    table_hbm.at[idx_v.at[b, j]],
                    rows_v.at[b, pl.ds(j * 128, 128)], gsem))
            for d in descs:
                d.wait()
            pltpu.async_copy(rows_v.at[b], out_hbm.at[pl.ds(e0, CH)],
                             wsem[b])
            if not tail:
                @pl.when(ci + 2 < nch)
                def _():
                    start_idx(ci + 2, b)

        start_idx(0, 0)
        start_idx(1, 1)

        def body(g, _):
            for b in range(2):
                process(g * 2 + b, b, False)
            return 0

        lax.fori_loop(0, nch // 2, body, 0)
        if nch % 2:
            process(nch - 1, (nch - 1) % 2, True)
        drain_out(nch % 2)
        drain_out(1 - nch % 2)

    return k(table, idx2d)


def _sc_scatter(msg128, dst2d):
    @functools.partial(
        pl.kernel, mesh=_mesh,
        out_type=jax.ShapeDtypeStruct((N, DIM), jnp.float32),
        compiler_params=pltpu.CompilerParams(use_tc_tiling_on_sc=False),
        scratch_types=[
            pltpu.VMEM((2, 8, 128), jnp.int32),
            pltpu.VMEM((8, 128), jnp.int32),
            pltpu.VMEM((2, CH, HD), jnp.float32),
            pltpu.VMEM_SHARED((SH, HD), jnp.float32),
            pltpu.SemaphoreType.DMA,
            pltpu.SemaphoreType.DMA,
            pltpu.SemaphoreType.DMA,
            pltpu.SemaphoreType.DMA,
            pltpu.SemaphoreType.DMA,
        ],
    )
    def k(msg_hbm, dst_hbm, out_hbm, raw_v, idx_v, val_v, shared,
          ssem, rsem0, rsem1, msem0, msem1):
        c = lax.axis_index("c")
        s = lax.axis_index("s")
        rsem = [rsem0, rsem1]
        msem = [msem0, msem1]
        nchp = EPS // CH
        col0 = c * HD

        zer = jnp.zeros((16,), jnp.float32)

        def zfill(i, _):
            val_v[0, i, pl.ds(0, 16)] = zer
            return 0

        def start_loads(ci, b):
            e0 = s * EPS + ci * CH
            r0 = pl.multiple_of(e0 // 128, 8)
            pltpu.async_copy(dst_hbm.at[pl.ds(r0, 8)], raw_v.at[b], rsem[b])
            pltpu.async_copy(msg_hbm.at[pl.ds(e0, CH), pl.ds(col0, HD)],
                             val_v.at[b], msem[b])

        def drain_loads(b):
            pltpu.make_async_copy(dst_hbm.at[pl.ds(0, 8)],
                                  raw_v.at[b], rsem[b]).wait()
            pltpu.make_async_copy(msg_hbm.at[pl.ds(0, CH), pl.ds(col0, HD)],
                                  val_v.at[b], msem[b]).wait()

        for p in range(2):
            lax.fori_loop(0, CH, zfill, 0)
            z0 = s * (SH // NS)
            done = 0
            while done < SH // NS:
                sz = min(CH, SH // NS - done)
                pltpu.sync_copy(val_v.at[0, pl.ds(0, sz)],
                                shared.at[pl.ds(z0 + done, sz)])
                done += sz
            plsc.subcore_barrier()

            lo = p * NH
            start_loads(0, 0)
            start_loads(1, 1)

            def body2(g, _):
                for b in range(2):
                    ci = g * 2 + b
                    drain_loads(b)
                    for j in range(8):
                        for m in range(8):
                            v = raw_v[b, j, pl.ds(m * 16, 16)]
                            rel = v - lo
                            ok = (rel >= 0) & (rel < NH)
                            idx_v[j, pl.ds(m * 16, 16)] = jnp.where(
                                ok, rel, TRASH)
                    descs = []
                    for j in range(8):
                        descs.append(pltpu.async_copy(
                            val_v.at[b, pl.ds(j * 128, 128)],
                            shared.at[idx_v.at[j]], ssem, add=True))
                    for d in descs:
                        d.wait()

                    @pl.when(ci + 2 < nchp)
                    def _():
                        start_loads(ci + 2, b)
                return 0

            lax.fori_loop(0, nchp // 2, body2, 0)
            plsc.subcore_barrier()

            rows = NH // NS
            pltpu.sync_copy(
                shared.at[pl.ds(s * rows, rows)],
                out_hbm.at[pl.ds(lo + s * rows, rows), pl.ds(col0, HD)])
            plsc.subcore_barrier()

    return k(msg128, dst2d)


def _tc_edge(ea_t, xs, w1, b1, w2p, b2p, rmat, smat, inp, grid):
    T = 1024
    G = grid

    def body(ea_ref, xs_ref, w1_ref, b1_ref, w2p_ref, b2p_ref, r_ref, s_ref,
             out_ref):
        ea = ea_ref[...]
        xsv = xs_ref[...]
        h = jnp.maximum(
            jax.lax.dot_general(ea, w1_ref[...], (((0,), (0,)), ((), ())),
                                preferred_element_type=jnp.float32)
            + b1_ref[...], 0.0)
        hq = jnp.dot(h, r_ref[...], preferred_element_type=jnp.float32)
        xq = jnp.dot(xsv, s_ref[...], preferred_element_type=jnp.float32)
        q = hq * xq
        out_ref[:, :DIM] = (
            jnp.dot(q, w2p_ref[...], preferred_element_type=jnp.float32)
            + jnp.dot(xsv, b2p_ref[...], preferred_element_type=jnp.float32))

    return pl.pallas_call(
        body,
        grid=(G,),
        in_specs=[
            pl.BlockSpec((EF, T), lambda i: (0, i)),
            pl.BlockSpec((T, inp), lambda i: (i, 0)),
            pl.BlockSpec((EF, DIM), lambda i: (0, 0)),
            pl.BlockSpec((1, DIM), lambda i: (0, 0)),
            pl.BlockSpec((DIM * inp, DIM), lambda i: (0, 0)),
            pl.BlockSpec((inp, DIM), lambda i: (0, 0)),
            pl.BlockSpec((DIM, DIM * inp), lambda i: (0, 0)),
            pl.BlockSpec((inp, DIM * inp), lambda i: (0, 0)),
        ],
        out_specs=pl.BlockSpec((T, 128), lambda i: (i, 0)),
        out_shape=jax.ShapeDtypeStruct((EH, 128), jnp.float32),
    )(ea_t, xs, w1, b1, w2p, b2p, rmat, smat)


def _tc_node(agg_a, agg_b, xtab, rootp, bias, inp):
    T = 1000
    G = N // T

    def body(aa_ref, ab_ref, x_ref, r_ref, b_ref, out_ref):
        out_ref[...] = jnp.maximum(
            aa_ref[...] + ab_ref[...]
            + jnp.dot(x_ref[...], r_ref[...], preferred_element_type=jnp.float32)
            + b_ref[...], 0.0)

    return pl.pallas_call(
        body,
        grid=(G,),
        in_specs=[
            pl.BlockSpec((T, DIM), lambda i: (i, 0)),
            pl.BlockSpec((T, DIM), lambda i: (i, 0)),
            pl.BlockSpec((T, inp), lambda i: (i, 0)),
            pl.BlockSpec((inp, DIM), lambda i: (0, 0)),
            pl.BlockSpec((1, DIM), lambda i: (0, 0)),
        ],
        out_specs=pl.BlockSpec((T, DIM), lambda i: (i, 0)),
        out_shape=jax.ShapeDtypeStruct((N, DIM), jnp.float32),
    )(agg_a, agg_b, xtab, rootp, bias)


def kernel(x, edge_index, edge_attr, batch,
           mlp1_w1, mlp1_b1, mlp1_w2, mlp1_b2, root1, bias1,
           mlp2_w1, mlp2_b1, mlp2_w2, mlp2_b2, root2, bias2):
    del batch
    src = edge_index[0].astype(jnp.int32)
    dst = edge_index[1].astype(jnp.int32)
    src2d = jnp.pad(src, (0, E_PAD - E)).reshape(E_PAD // 128, 128)
    dst2d = jnp.pad(dst, (0, E_PAD - E),
                    constant_values=N).reshape(E_PAD // 128, 128)
    ea_t = jnp.pad(edge_attr.T, ((0, 0), (0, 1024 * 1563 - E)))

    xpad = jnp.pad(x, ((0, 0), (0, 16 - NF)))
    w2r1 = mlp1_w2.reshape(DIM, NF, DIM)
    w2p1 = jnp.pad(w2r1, ((0, 0), (0, 16 - NF), (0, 0))).reshape(DIM * 16, DIM)
    b2p1 = jnp.pad(mlp1_b2.reshape(NF, DIM), ((0, 16 - NF), (0, 0)))
    root1p = jnp.pad(root1, ((0, 16 - NF), (0, 0)))

    w2p2 = mlp2_w2.reshape(DIM * DIM, DIM)
    b2p2 = mlp2_b2.reshape(DIM, DIM)

    halves = (
        (src2d[:EH // 128], dst2d[:EH // 128], ea_t[:, :EH], EH // 1024),
        (src2d[EH // 128:], dst2d[EH // 128:], ea_t[:, EH:],
         (1024 * 1563 - EH) // 1024),
    )

    def layer(xtab, w1, b1, w2p, b2p, rootp, bias, inp):
        rmat = jnp.repeat(jnp.eye(DIM, dtype=jnp.float32), inp, axis=1)
        smat = jnp.tile(jnp.eye(inp, dtype=jnp.float32), (1, DIM))
        aggs = []
        for s2, d2, eat, grid in halves:
            xs = _sc_gather(xtab, s2, inp)
            msg = _tc_edge(eat, xs, w1, b1.reshape(1, DIM), w2p, b2p,
                           rmat, smat, inp, grid)
            aggs.append(_sc_scatter(msg, d2))
        return _tc_node(aggs[0], aggs[1], xtab, rootp, bias.reshape(1, DIM),
                        inp)

    x1 = layer(xpad, mlp1_w1, mlp1_b1, w2p1, b2p1, root1p, bias1, 16)
    x2 = layer(x1, mlp2_w1, mlp2_b1, w2p2, b2p2, root2, bias2, DIM)
    return x2


# speedup vs baseline: 5.5926x; 1.2119x over previous
import functools

import jax
import jax.numpy as jnp
from jax import lax
from jax.experimental import pallas as pl
from jax.experimental.pallas import tpu as pltpu
from jax.experimental.pallas import tpu_sc as plsc

N = 100000
E = 1600000
DIM = 32
NF = 11
EF = 4

NC = 2
NS = 16
NW = NC * NS

E_PAD = 1638400
EH = E_PAD // 2
CH = 1024
CHS = 512
EPW = EH // NW
EPS = EH // NS
SH = N + 16
HD = DIM // 2

_mesh = plsc.VectorSubcoreMesh(core_axis_name="c", subcore_axis_name="s",
                               num_cores=NC, num_subcores=NS)


def _sc_gather(table, idx2d, F):
    @functools.partial(
        pl.kernel, mesh=_mesh,
        out_type=jax.ShapeDtypeStruct((EH, F), jnp.float32),
        compiler_params=pltpu.CompilerParams(use_tc_tiling_on_sc=False),
        scratch_types=[
            pltpu.VMEM((2, 8, 128), jnp.int32),
            pltpu.VMEM((2, CH, F), jnp.float32),
            pltpu.SemaphoreType.DMA,
            pltpu.SemaphoreType.DMA,
            pltpu.SemaphoreType.DMA,
            pltpu.SemaphoreType.DMA,
            pltpu.SemaphoreType.DMA,
        ],
    )
    def k(table_hbm, idx_hbm, out_hbm, idx_v, rows_v, gsem,
          isem0, isem1, wsem0, wsem1):
        isem = [isem0, isem1]
        wsem = [wsem0, wsem1]
        wid = lax.axis_index("s") * NC + lax.axis_index("c")
        nch = EPW // CH

        def start_idx(ci, b):
            e0 = wid * EPW + ci * CH
            r0 = pl.multiple_of(e0 // 128, 8)
            pltpu.async_copy(idx_hbm.at[pl.ds(r0, 8)], idx_v.at[b], isem[b])

        def drain_idx(b):
            pltpu.make_async_copy(idx_hbm.at[pl.ds(0, 8)],
                                  idx_v.at[b], isem[b]).wait()

        def drain_out(b):
            pltpu.make_async_copy(rows_v.at[b],
                                  out_hbm.at[pl.ds(0, CH)], wsem[b]).wait()

        def process(ci, b, tail):
            e0 = wid * EPW + ci * CH
            drain_idx(b)
            if tail:
                drain_out(b)
            else:
                @pl.when(ci >= 2)
                def _():
                    drain_out(b)
            descs = []
            for j in range(8):
                descs.append(pltpu.async_copy(
                    table_hbm.at[idx_v.at[b, j]],
                    rows_v.at[b, pl.ds(j * 128, 128)], gsem))
            for d in descs:
                d.wait()
            pltpu.async_copy(rows_v.at[b], out_hbm.at[pl.ds(e0, CH)],
                             wsem[b])
            if not tail:
                @pl.when(ci + 2 < nch)
                def _():
                    start_idx(ci + 2, b)

        start_idx(0, 0)
        start_idx(1, 1)

        def body(g, _):
            for b in range(2):
                process(g * 2 + b, b, False)
            return 0

        lax.fori_loop(0, nch // 2, body, 0)
        if nch % 2:
            process(nch - 1, (nch - 1) % 2, True)
        drain_out(nch % 2)
        drain_out(1 - nch % 2)

    return k(table, idx2d)


def _sc_scatter(msg128, dst2d):
    @functools.partial(
        pl.kernel, mesh=_mesh,
        out_type=jax.ShapeDtypeStruct((N, DIM), jnp.float32),
        compiler_params=pltpu.CompilerParams(use_tc_tiling_on_sc=False),
        scratch_types=[
            pltpu.VMEM((2, 4, 128), jnp.int32),
            pltpu.VMEM((2, CHS, HD), jnp.float32),
            pltpu.VMEM_SHARED((SH, HD), jnp.float32),
            pltpu.SemaphoreType.DMA,
            pltpu.SemaphoreType.DMA,
            pltpu.SemaphoreType.DMA,
            pltpu.SemaphoreType.DMA,
            pltpu.SemaphoreType.DMA,
        ],
    )
    def k(msg_hbm, dst_hbm, out_hbm, raw_v, val_v, shared,
          ssem, rsem0, rsem1, msem0, msem1):
        c = lax.axis_index("c")
        s = lax.axis_index("s")
        rsem = [rsem0, rsem1]
        msem = [msem0, msem1]
        nchp = EPS // CHS
        col0 = c * HD

        zer = jnp.zeros((16,), jnp.float32)

        def zfill(i, _):
            val_v[0, i, pl.ds(0, 16)] = zer
            return 0

        def start_loads(ci, b):
            e0 = s * EPS + ci * CHS
            r0 = pl.multiple_of(e0 // 128, 4)
            pltpu.async_copy(dst_hbm.at[pl.ds(r0, 4)], raw_v.at[b], rsem[b])
            pltpu.async_copy(msg_hbm.at[pl.ds(e0, CHS), pl.ds(col0, HD)],
                             val_v.at[b], msem[b])

        def drain_loads(b):
            pltpu.make_async_copy(dst_hbm.at[pl.ds(0, 4)],
                                  raw_v.at[b], rsem[b]).wait()
            pltpu.make_async_copy(msg_hbm.at[pl.ds(0, CHS), pl.ds(col0, HD)],
                                  val_v.at[b], msem[b]).wait()

        lax.fori_loop(0, CHS, zfill, 0)
        z0 = s * (SH // NS)
        done = 0
        while done < SH // NS:
            sz = min(CHS, SH // NS - done)
            pltpu.sync_copy(val_v.at[0, pl.ds(0, sz)],
                            shared.at[pl.ds(z0 + done, sz)])
            done += sz
        plsc.subcore_barrier()

        start_loads(0, 0)
        start_loads(1, 1)

        def body2(g, _):
            for b in range(2):
                ci = g * 2 + b
                drain_loads(b)
                descs = []
                for j in range(4):
                    descs.append(pltpu.async_copy(
                        val_v.at[b, pl.ds(j * 128, 128)],
                        shared.at[raw_v.at[b, j]], ssem, add=True))
                for d in descs:
                    d.wait()

                @pl.when(ci + 2 < nchp)
                def _():
                    start_loads(ci + 2, b)
            return 0

        lax.fori_loop(0, nchp // 2, body2, 0)
        plsc.subcore_barrier()

        rows = N // NS
        pltpu.sync_copy(
            shared.at[pl.ds(s * rows, rows)],
            out_hbm.at[pl.ds(s * rows, rows), pl.ds(col0, HD)])

    return k(msg128, dst2d)


def _tc_edge(ea_t, xs, w1, b1, w2p, b2p, rmat, smat, inp, grid):
    T = 1024
    G = grid

    def body(ea_ref, xs_ref, w1_ref, b1_ref, w2p_ref, b2p_ref, r_ref, s_ref,
             out_ref):
        ea = ea_ref[...]
        xsv = xs_ref[...]
        h = jnp.maximum(
            jax.lax.dot_general(ea, w1_ref[...], (((0,), (0,)), ((), ())),
                                preferred_element_type=jnp.float32)
            + b1_ref[...], 0.0)
        hq = jnp.dot(h, r_ref[...], preferred_element_type=jnp.float32)
        xq = jnp.dot(xsv, s_ref[...], preferred_element_type=jnp.float32)
        q = hq * xq
        out_ref[:, :DIM] = (
            jnp.dot(q, w2p_ref[...], preferred_element_type=jnp.float32)
            + jnp.dot(xsv, b2p_ref[...], preferred_element_type=jnp.float32))

    return pl.pallas_call(
        body,
        grid=(G,),
        in_specs=[
            pl.BlockSpec((EF, T), lambda i: (0, i)),
            pl.BlockSpec((T, inp), lambda i: (i, 0)),
            pl.BlockSpec((EF, DIM), lambda i: (0, 0)),
            pl.BlockSpec((1, DIM), lambda i: (0, 0)),
            pl.BlockSpec((DIM * inp, DIM), lambda i: (0, 0)),
            pl.BlockSpec((inp, DIM), lambda i: (0, 0)),
            pl.BlockSpec((DIM, DIM * inp), lambda i: (0, 0)),
            pl.BlockSpec((inp, DIM * inp), lambda i: (0, 0)),
        ],
        out_specs=pl.BlockSpec((T, 128), lambda i: (i, 0)),
        out_shape=jax.ShapeDtypeStruct((EH, 128), jnp.float32),
    )(ea_t, xs, w1, b1, w2p, b2p, rmat, smat)


def _tc_node(agg_a, agg_b, xtab, rootp, bias, inp):
    T = 1000
    G = N // T

    def body(aa_ref, ab_ref, x_ref, r_ref, b_ref, out_ref):
        out_ref[...] = jnp.maximum(
            aa_ref[...] + ab_ref[...]
            + jnp.dot(x_ref[...], r_ref[...], preferred_element_type=jnp.float32)
            + b_ref[...], 0.0)

    return pl.pallas_call(
        body,
        grid=(G,),
        in_specs=[
            pl.BlockSpec((T, DIM), lambda i: (i, 0)),
            pl.BlockSpec((T, DIM), lambda i: (i, 0)),
            pl.BlockSpec((T, inp), lambda i: (i, 0)),
            pl.BlockSpec((inp, DIM), lambda i: (0, 0)),
            pl.BlockSpec((1, DIM), lambda i: (0, 0)),
        ],
        out_specs=pl.BlockSpec((T, DIM), lambda i: (i, 0)),
        out_shape=jax.ShapeDtypeStruct((N, DIM), jnp.float32),
    )(agg_a, agg_b, xtab, rootp, bias)


def kernel(x, edge_index, edge_attr, batch,
           mlp1_w1, mlp1_b1, mlp1_w2, mlp1_b2, root1, bias1,
           mlp2_w1, mlp2_b1, mlp2_w2, mlp2_b2, root2, bias2):
    del batch
    src = edge_index[0].astype(jnp.int32)
    dst = edge_index[1].astype(jnp.int32)
    src2d = jnp.pad(src, (0, E_PAD - E)).reshape(E_PAD // 128, 128)
    dst2d = jnp.pad(dst, (0, E_PAD - E),
                    constant_values=N).reshape(E_PAD // 128, 128)
    ea_t = jnp.pad(edge_attr.T, ((0, 0), (0, 1024 * 1563 - E)))

    xpad = jnp.pad(x, ((0, 0), (0, 16 - NF)))
    w2r1 = mlp1_w2.reshape(DIM, NF, DIM)
    w2p1 = jnp.pad(w2r1, ((0, 0), (0, 16 - NF), (0, 0))).reshape(DIM * 16, DIM)
    b2p1 = jnp.pad(mlp1_b2.reshape(NF, DIM), ((0, 16 - NF), (0, 0)))
    root1p = jnp.pad(root1, ((0, 16 - NF), (0, 0)))

    w2p2 = mlp2_w2.reshape(DIM * DIM, DIM)
    b2p2 = mlp2_b2.reshape(DIM, DIM)

    halves = (
        (src2d[:EH // 128], dst2d[:EH // 128], ea_t[:, :EH], EH // 1024),
        (src2d[EH // 128:], dst2d[EH // 128:], ea_t[:, EH:],
         (1024 * 1563 - EH) // 1024),
    )

    def layer(xtab, w1, b1, w2p, b2p, rootp, bias, inp):
        rmat = jnp.repeat(jnp.eye(DIM, dtype=jnp.float32), inp, axis=1)
        smat = jnp.tile(jnp.eye(inp, dtype=jnp.float32), (1, DIM))
        aggs = []
        for s2, d2, eat, grid in halves:
            xs = _sc_gather(xtab, s2, inp)
            msg = _tc_edge(eat, xs, w1, b1.reshape(1, DIM), w2p, b2p,
                           rmat, smat, inp, grid)
            aggs.append(_sc_scatter(msg, d2))
        return _tc_node(aggs[0], aggs[1], xtab, rootp, bias.reshape(1, DIM),
                        inp)

    x1 = layer(xpad, mlp1_w1, mlp1_b1, w2p1, b2p1, root1p, bias1, 16)
    x2 = layer(x1, mlp2_w1, mlp2_b1, w2p2, b2p2, root2, bias2, DIM)
    return x2
